# baseline (speedup 1.0000x reference)
import functools

import jax
import jax.numpy as jnp
from jax import lax
from jax.experimental import pallas as pl
from jax.experimental.pallas import tpu as pltpu
from jax.experimental.pallas import tpu_sc as plsc

_LANES = 16


@functools.lru_cache(maxsize=None)
def _make_scale_kernel(total_words: int, scale: float):
    info = plsc.get_sparse_core_info()
    num_workers = info.num_cores * info.num_subcores
    assert total_words % num_workers == 0
    words_per_worker = total_words // num_workers
    chunk = min(words_per_worker, 64 * 1024)
    n_chunks = words_per_worker // chunk

    mesh = plsc.VectorSubcoreMesh(core_axis_name="c", subcore_axis_name="s")

    @functools.partial(
        pl.kernel,
        mesh=mesh,
        out_type=jax.ShapeDtypeStruct((total_words,), jnp.float32),
        scratch_types=[
            pltpu.VMEM((chunk,), jnp.float32),
        ],
    )
    def scale_kernel(emb_hbm, out_hbm, buf):
        wid = lax.axis_index("s") * info.num_cores + lax.axis_index("c")
        base = wid * words_per_worker

        def chunk_body(c, _):
            off = base + c * chunk
            pltpu.sync_copy(emb_hbm.at[pl.ds(off, chunk)], buf)

            @plsc.parallel_loop(0, chunk, step=_LANES, unroll=8)
            def _(i):
                buf[pl.ds(i, _LANES)] = buf[pl.ds(i, _LANES)] * scale

            pltpu.sync_copy(buf, out_hbm.at[pl.ds(off, chunk)])
            return 0

        lax.fori_loop(0, n_chunks, chunk_body, 0)

    return scale_kernel


def kernel(x, emb_weight):
    seq_len = x.shape[1]
    dim = emb_weight.shape[1]
    scale = dim ** -0.5
    flat = emb_weight[:seq_len].reshape(-1)
    out = _make_scale_kernel(flat.shape[0], scale)(flat)
    return out.reshape(seq_len, dim)


# speedup vs baseline: 1.0741x; 1.0741x over previous
import functools

import jax
import jax.numpy as jnp
from jax import lax
from jax.experimental import pallas as pl
from jax.experimental.pallas import tpu as pltpu
from jax.experimental.pallas import tpu_sc as plsc

_LANES = 16


@functools.lru_cache(maxsize=None)
def _make_scale_kernel(total_words: int, scale: float):
    info = plsc.get_sparse_core_info()
    num_workers = info.num_cores * info.num_subcores
    assert total_words % num_workers == 0
    words_per_worker = total_words // num_workers
    chunk = min(words_per_worker, 32 * 1024)
    n_chunks = words_per_worker // chunk

    mesh = plsc.VectorSubcoreMesh(core_axis_name="c", subcore_axis_name="s")

    @functools.partial(
        pl.kernel,
        mesh=mesh,
        out_type=jax.ShapeDtypeStruct((total_words,), jnp.float32),
        scratch_types=[
            pltpu.VMEM((chunk,), jnp.float32),
            pltpu.VMEM((chunk,), jnp.float32),
            pltpu.SemaphoreType.DMA,
            pltpu.SemaphoreType.DMA,
            pltpu.SemaphoreType.DMA,
            pltpu.SemaphoreType.DMA,
        ],
    )
    def scale_kernel(emb_hbm, out_hbm, buf0, buf1, si0, si1, so0, so1):
        wid = lax.axis_index("s") * info.num_cores + lax.axis_index("c")
        base = wid * words_per_worker
        bufs = (buf0, buf1)
        sin = (si0, si1)
        sout = (so0, so1)

        in_copies = [None, None]
        out_copies = [None, None]
        in_copies[0] = pltpu.async_copy(
            emb_hbm.at[pl.ds(base, chunk)], buf0, si0)
        for c in range(n_chunks):
            b = c % 2
            nb = (c + 1) % 2
            if c + 1 < n_chunks:
                if out_copies[nb] is not None:
                    out_copies[nb].wait()
                in_copies[nb] = pltpu.async_copy(
                    emb_hbm.at[pl.ds(base + (c + 1) * chunk, chunk)],
                    bufs[nb], sin[nb])
            in_copies[b].wait()
            buf = bufs[b]

            @plsc.parallel_loop(0, chunk, step=_LANES, unroll=8)
            def _(i):
                buf[pl.ds(i, _LANES)] = buf[pl.ds(i, _LANES)] * scale

            out_copies[b] = pltpu.async_copy(
                buf, out_hbm.at[pl.ds(base + c * chunk, chunk)], sout[b])
        out_copies[(n_chunks - 2) % 2].wait()
        out_copies[(n_chunks - 1) % 2].wait()

    return scale_kernel


def kernel(x, emb_weight):
    seq_len = x.shape[1]
    dim = emb_weight.shape[1]
    scale = dim ** -0.5
    flat = emb_weight[:seq_len].reshape(-1)
    out = _make_scale_kernel(flat.shape[0], scale)(flat)
    return out.reshape(seq_len, dim)


# speedup vs baseline: 2.5593x; 2.3827x over previous
import functools

import jax
import jax.numpy as jnp
from jax import lax
from jax.experimental import pallas as pl
from jax.experimental.pallas import tpu as pltpu
from jax.experimental.pallas import tpu_sc as plsc

_LANES = 16


@functools.lru_cache(maxsize=None)
def _make_scale_kernel(rows: int, dim: int, scale: float):
    info = plsc.get_sparse_core_info()
    num_workers = info.num_cores * info.num_subcores
    assert rows % num_workers == 0
    rows_per_worker = rows // num_workers
    chunk_rows = min(rows_per_worker, 32)
    n_chunks = rows_per_worker // chunk_rows

    mesh = plsc.VectorSubcoreMesh(core_axis_name="c", subcore_axis_name="s")

    @functools.partial(
        pl.kernel,
        mesh=mesh,
        out_type=jax.ShapeDtypeStruct((rows, dim), jnp.float32),
        scratch_types=[
            pltpu.VMEM((chunk_rows, dim), jnp.float32),
            pltpu.VMEM((chunk_rows, dim), jnp.float32),
            pltpu.SemaphoreType.DMA,
            pltpu.SemaphoreType.DMA,
            pltpu.SemaphoreType.DMA,
            pltpu.SemaphoreType.DMA,
        ],
        compiler_params=pltpu.CompilerParams(use_tc_tiling_on_sc=True),
    )
    def scale_kernel(emb_hbm, out_hbm, buf0, buf1, si0, si1, so0, so1):
        wid = lax.axis_index("s") * info.num_cores + lax.axis_index("c")
        base = wid * rows_per_worker
        bufs = (buf0, buf1)
        sin = (si0, si1)
        sout = (so0, so1)

        in_copies = [None, None]
        out_copies = [None, None]
        in_copies[0] = pltpu.async_copy(
            emb_hbm.at[pl.ds(base, chunk_rows)], buf0, si0)
        for c in range(n_chunks):
            b = c % 2
            nb = (c + 1) % 2
            if c + 1 < n_chunks:
                if out_copies[nb] is not None:
                    out_copies[nb].wait()
                in_copies[nb] = pltpu.async_copy(
                    emb_hbm.at[pl.ds(base + (c + 1) * chunk_rows, chunk_rows)],
                    bufs[nb], sin[nb])
            in_copies[b].wait()
            buf = bufs[b]

            def row_body(r, _):
                @plsc.parallel_loop(0, dim, step=_LANES, unroll=8)
                def _scale(i):
                    buf[r, pl.ds(i, _LANES)] = buf[r, pl.ds(i, _LANES)] * scale

                return 0

            lax.fori_loop(0, chunk_rows, row_body, 0)

            out_copies[b] = pltpu.async_copy(
                buf, out_hbm.at[pl.ds(base + c * chunk_rows, chunk_rows)],
                sout[b])
        out_copies[(n_chunks - 2) % 2].wait()
        out_copies[(n_chunks - 1) % 2].wait()

    return scale_kernel


def kernel(x, emb_weight):
    seq_len = x.shape[1]
    dim = emb_weight.shape[1]
    scale = dim ** -0.5
    return _make_scale_kernel(seq_len, dim, scale)(emb_weight[:seq_len])
